# baseline (speedup 1.0000x reference)
import functools

import jax
import jax.numpy as jnp
from jax import lax
from jax.experimental import pallas as pl
from jax.experimental.pallas import tpu as pltpu
from jax.experimental.pallas import tpu_sc as plsc

N = 10000
E = 160000
EH = E // 2
D = 256
HD = D // 2
NB = 1000
EB = 1000


@functools.cache
def _sc_mesh():
    return plsc.VectorSubcoreMesh(core_axis_name="c", subcore_axis_name="s")


def _swish(x):
    return x * jax.nn.sigmoid(x)


def _node_pre_body(h_ref, wa_ref, wb_ref, ga_ref, gb_ref):
    x = h_ref[:]
    ga_ref[:] = jnp.dot(x, wa_ref[:], preferred_element_type=jnp.float32)
    gb_ref[:] = jnp.dot(x, wb_ref[:], preferred_element_type=jnp.float32)


def _node_pre(h, wa, wb):
    grid = N // NB
    return pl.pallas_call(
        _node_pre_body,
        grid=(grid,),
        in_specs=[
            pl.BlockSpec((NB, D), lambda i: (i, 0)),
            pl.BlockSpec((D, D), lambda i: (0, 0)),
            pl.BlockSpec((D, D), lambda i: (0, 0)),
        ],
        out_specs=[
            pl.BlockSpec((NB, D), lambda i: (i, 0)),
            pl.BlockSpec((NB, D), lambda i: (i, 0)),
        ],
        out_shape=[
            jax.ShapeDtypeStruct((N, D), jnp.float32),
            jax.ShapeDtypeStruct((N, D), jnp.float32),
        ],
    )(h, wa, wb)


LG = 80
NCG = EH // LG
NCW = 32


@functools.cache
def _gather_kernel():
    return pl.kernel(
        _gather_body,
        out_type=jax.ShapeDtypeStruct((EH, D), jnp.float32),
        mesh=_sc_mesh(),
        scratch_types=[
            pltpu.VMEM((NCW * LG,), jnp.int32),
            pltpu.VMEM((NCW * LG,), jnp.int32),
            pltpu.VMEM((LG, D), jnp.float32),
            pltpu.VMEM((LG, D), jnp.float32),
            pltpu.VMEM((LG, D), jnp.float32),
            pltpu.VMEM((LG, D), jnp.float32),
            pltpu.SemaphoreType.DMA,
            pltpu.SemaphoreType.DMA,
            pltpu.SemaphoreType.DMA,
            pltpu.SemaphoreType.DMA,
            pltpu.SemaphoreType.DMA,
            pltpu.SemaphoreType.DMA,
        ],
    )


def _gather_body(ga, gb, ii, jj, out, idxa, idxb, ra0, ra1, rb0, rb1,
                 sa0, sa1, sb0, sb1, sw0, sw1):
    cid = lax.axis_index("c")
    sid = lax.axis_index("s")
    wid = sid * 2 + cid
    ebase = wid * (NCW * LG)
    nch = jnp.minimum(NCW, NCG - NCW * wid)

    @pl.when(wid < 31)
    def _():
        pltpu.sync_copy(ii.at[pl.ds(ebase, NCW * LG)], idxa)
        pltpu.sync_copy(jj.at[pl.ds(ebase, NCW * LG)], idxb)

    @pl.when(wid == 31)
    def _():
        tail = (NCG - NCW * 31) * LG
        pltpu.sync_copy(ii.at[pl.ds(ebase, tail)], idxa.at[pl.ds(0, tail)])
        pltpu.sync_copy(jj.at[pl.ds(ebase, tail)], idxb.at[pl.ds(0, tail)])

    ra = (ra0, ra1)
    rb = (rb0, rb1)
    sa = (sa0, sa1)
    sb = (sb0, sb1)
    sw = (sw0, sw1)

    def issue(k, par):
        sl = pl.ds(k * LG, LG)
        pltpu.async_copy(ga.at[idxa.at[sl]], ra[par], sa[par])
        pltpu.async_copy(gb.at[idxb.at[sl]], rb[par], sb[par])

    def wait_gathers(k, par):
        sl = pl.ds(k * LG, LG)
        pltpu.make_async_copy(ga.at[idxa.at[sl]], ra[par], sa[par]).wait()
        pltpu.make_async_copy(gb.at[idxb.at[sl]], rb[par], sb[par]).wait()

    def add(par):
        def row(r, c2):
            for c in range(D // 16):
                sl = pl.ds(c * 16, 16)
                plsc.addupdate(ra[par].at[r, sl], rb[par][r, sl])
            return c2

        lax.fori_loop(0, LG, row, 0)

    def issue_wb(k, par):
        pltpu.async_copy(ra[par], out.at[pl.ds(ebase + k * LG, LG)], sw[par])

    def wait_wb(k, par):
        pltpu.make_async_copy(
            ra[par], out.at[pl.ds(ebase + k * LG, LG)], sw[par]).wait()

    issue(0, 0)

    def pair(p, carry):
        k0 = 2 * p
        @pl.when(p > 0)
        def _():
            wait_wb(k0 - 1, 1)

        issue(k0 + 1, 1)
        wait_gathers(k0, 0)
        add(0)
        issue_wb(k0, 0)
        @pl.when(p < nch // 2 - 1)
        def _():
            wait_wb(k0, 0)
            issue(k0 + 2, 0)

        wait_gathers(k0 + 1, 1)
        add(1)
        issue_wb(k0 + 1, 1)
        return carry

    lax.fori_loop(0, nch // 2, pair, 0)
    wait_wb(nch - 2, 0)
    wait_wb(nch - 1, 1)


def _edge_mlp_body(s_ref, ea_ref, w1c_ref, b1_ref, w2_ref, b2_ref, out_ref):
    z = s_ref[:] + jnp.dot(ea_ref[:], w1c_ref[:],
                           preferred_element_type=jnp.float32) + b1_ref[:]
    z = _swish(z)
    m = jnp.dot(z, w2_ref[:], preferred_element_type=jnp.float32) + b2_ref[:]
    out_ref[:] = _swish(m)


def _edge_mlp(s, ea, w1c, b1, w2, b2, off):
    grid = EH // EB
    return pl.pallas_call(
        _edge_mlp_body,
        grid=(grid,),
        in_specs=[
            pl.BlockSpec((EB, D), lambda i: (i, 0)),
            pl.BlockSpec((EB, D), lambda i, off=off: (i + off, 0)),
            pl.BlockSpec((D, D), lambda i: (0, 0)),
            pl.BlockSpec((1, D), lambda i: (0, 0)),
            pl.BlockSpec((D, D), lambda i: (0, 0)),
            pl.BlockSpec((1, D), lambda i: (0, 0)),
        ],
        out_specs=pl.BlockSpec((EB, D), lambda i: (i, 0)),
        out_shape=jax.ShapeDtypeStruct((EH, D), jnp.float32),
    )(s, ea, w1c, b1, w2, b2)


NPAD = 10240
_RPT = NPAD // 16
LD = 128
NCD = EH // LD
NCT = 40


@functools.cache
def _scatter_kernel():
    return pl.kernel(
        _scatter_body,
        out_type=jax.ShapeDtypeStruct((NPAD, D), jnp.float32),
        mesh=_sc_mesh(),
        scratch_types=[
            pltpu.VMEM((LD,), jnp.int32),
            pltpu.VMEM((LD,), jnp.int32),
            pltpu.VMEM((LD, HD), jnp.float32),
            pltpu.VMEM((LD, HD), jnp.float32),
            pltpu.VMEM_SHARED((NPAD, HD), jnp.float32),
            pltpu.SemaphoreType.DMA,
            pltpu.SemaphoreType.DMA,
            pltpu.SemaphoreType.DMA,
            pltpu.SemaphoreType.DMA,
        ],
    )


def _scatter_body(mij, ii, out, idx0, idx1, buf0, buf1, acc,
                  sl0, sl1, si0, si1):
    cid = lax.axis_index("c")
    sid = lax.axis_index("s")
    f0 = cid * HD
    base = sid * NCT
    nch = jnp.minimum(NCT, NCD - base)

    zv = jnp.zeros((16,), jnp.float32)

    def zrow(r, c2):
        for c in range(HD // 16):
            buf0[r, pl.ds(c * 16, 16)] = zv
        return c2

    lax.fori_loop(0, LD, zrow, 0)
    r0 = sid * _RPT
    for k in range(_RPT // LD):
        pltpu.sync_copy(buf0, acc.at[pl.ds(r0 + k * LD, LD)])
    plsc.subcore_barrier()

    bufs = (buf0, buf1)
    idxs = (idx0, idx1)
    sls = (sl0, sl1)
    sis = (si0, si1)

    def issue(k, par):
        e0 = (base + k) * LD
        pltpu.async_copy(mij.at[pl.ds(e0, LD), pl.ds(f0, HD)],
                         bufs[par], sls[par])
        pltpu.async_copy(ii.at[pl.ds(e0, LD)], idxs[par], sis[par])

    def wait_load(k, par):
        e0 = (base + k) * LD
        pltpu.make_async_copy(mij.at[pl.ds(e0, LD), pl.ds(f0, HD)],
                              bufs[par], sls[par]).wait()
        pltpu.make_async_copy(ii.at[pl.ds(e0, LD)],
                              idxs[par], sis[par]).wait()

    def scatter_add(par):
        pltpu.sync_copy(bufs[par], acc.at[idxs[par]], add=True)

    issue(0, 0)

    def pair(p, carry):
        k0 = 2 * p
        issue(k0 + 1, 1)
        wait_load(k0, 0)
        scatter_add(0)
        @pl.when(k0 + 2 < nch)
        def _():
            issue(k0 + 2, 0)

        wait_load(k0 + 1, 1)
        scatter_add(1)
        return carry

    lax.fori_loop(0, nch // 2, pair, 0)

    @pl.when(nch % 2 == 1)
    def _():
        wait_load(nch - 1, 0)
        scatter_add(0)

    plsc.subcore_barrier()

    for k in range(_RPT // LD):
        b = bufs[k % 2]
        pltpu.sync_copy(acc.at[pl.ds(r0 + k * LD, LD)], b)
        pltpu.sync_copy(b, out.at[pl.ds(r0 + k * LD, LD), pl.ds(f0, HD)])


def _node_mlp_body(h_ref, a1_ref, a2_ref, w1a_ref, w1b_ref, b1_ref,
                   w2_ref, b2_ref, out_ref):
    hb = h_ref[:]
    agg = a1_ref[:] + a2_ref[:]
    t = (jnp.dot(hb, w1a_ref[:], preferred_element_type=jnp.float32)
         + jnp.dot(agg, w1b_ref[:], preferred_element_type=jnp.float32)
         + b1_ref[:])
    t = _swish(t)
    out_ref[:] = hb + jnp.dot(t, w2_ref[:],
                              preferred_element_type=jnp.float32) + b2_ref[:]


def _node_mlp(h, agg1, agg2, w1a, w1b, b1, w2, b2):
    grid = N // NB
    return pl.pallas_call(
        _node_mlp_body,
        grid=(grid,),
        in_specs=[
            pl.BlockSpec((NB, D), lambda i: (i, 0)),
            pl.BlockSpec((NB, D), lambda i: (i, 0)),
            pl.BlockSpec((NB, D), lambda i: (i, 0)),
            pl.BlockSpec((D, D), lambda i: (0, 0)),
            pl.BlockSpec((D, D), lambda i: (0, 0)),
            pl.BlockSpec((1, D), lambda i: (0, 0)),
            pl.BlockSpec((D, D), lambda i: (0, 0)),
            pl.BlockSpec((1, D), lambda i: (0, 0)),
        ],
        out_specs=pl.BlockSpec((NB, D), lambda i: (i, 0)),
        out_shape=jax.ShapeDtypeStruct((N, D), jnp.float32),
    )(h, agg1, agg2, w1a, w1b, b1, w2, b2)


def kernel(h, edge_index, edge_attr, e_w1, e_b1, e_w2, e_b2,
           n_w1, n_b1, n_w2, n_b2):
    ii = edge_index[0]
    jj = edge_index[1]
    ii1, ii2 = ii[:EH], ii[EH:]
    jj1, jj2 = jj[:EH], jj[EH:]
    w1a = e_w1[:D]
    w1b = e_w1[D:2 * D]
    w1c = e_w1[2 * D:]
    e_b1r = e_b1.reshape(1, D)
    e_b2r = e_b2.reshape(1, D)

    ga, gb = _node_pre(h, w1a, w1b)
    s1 = _gather_kernel()(ga, gb, ii1, jj1)
    s2 = _gather_kernel()(ga, gb, ii2, jj2)
    mij1 = _edge_mlp(s1, edge_attr, w1c, e_b1r, e_w2, e_b2r, 0)
    mij2 = _edge_mlp(s2, edge_attr, w1c, e_b1r, e_w2, e_b2r, EH // EB)
    agg1 = _scatter_kernel()(mij1, ii1)
    agg2 = _scatter_kernel()(mij2, ii2)
    h_out = _node_mlp(h, agg1, agg2, n_w1[:D], n_w1[D:], n_b1.reshape(1, D),
                      n_w2, n_b2.reshape(1, D))
    mij = jnp.concatenate([mij1, mij2], axis=0)
    return (h_out, mij)


# speedup vs baseline: 1.1419x; 1.1419x over previous
import functools

import jax
import jax.numpy as jnp
from jax import lax
from jax.experimental import pallas as pl
from jax.experimental.pallas import tpu as pltpu
from jax.experimental.pallas import tpu_sc as plsc

N = 10000
E = 160000
D = 256
HD = D // 2
L = 128
NCH = E // L
NB = 1000
EB = 2000

@functools.cache
def _sc_mesh():
    return plsc.VectorSubcoreMesh(core_axis_name="c", subcore_axis_name="s")


def _swish(x):
    return x * jax.nn.sigmoid(x)


def _node_pre_body(h_ref, wa_ref, wb_ref, ga_ref, gb_ref):
    x = h_ref[:]
    ga_ref[:] = jnp.dot(x, wa_ref[:], preferred_element_type=jnp.float32)
    gb_ref[:] = jnp.dot(x, wb_ref[:], preferred_element_type=jnp.float32)


def _node_pre(h, wa, wb):
    grid = N // NB
    return pl.pallas_call(
        _node_pre_body,
        grid=(grid,),
        in_specs=[
            pl.BlockSpec((NB, D), lambda i: (i, 0)),
            pl.BlockSpec((D, D), lambda i: (0, 0)),
            pl.BlockSpec((D, D), lambda i: (0, 0)),
        ],
        out_specs=[
            pl.BlockSpec((NB, D), lambda i: (i, 0)),
            pl.BlockSpec((NB, D), lambda i: (i, 0)),
        ],
        out_shape=[
            jax.ShapeDtypeStruct((N, D), jnp.float32),
            jax.ShapeDtypeStruct((N, D), jnp.float32),
        ],
    )(h, wa, wb)


LG = 80
NCG = E // LG
NCW = 64


@functools.cache
def _gather_kernel():
    return pl.kernel(
        _gather_body,
        out_type=jax.ShapeDtypeStruct((E, D), jnp.float32),
        mesh=_sc_mesh(),
        scratch_types=[
            pltpu.VMEM((NCW * LG,), jnp.int32),
            pltpu.VMEM((NCW * LG,), jnp.int32),
            pltpu.VMEM((LG, D), jnp.float32),
            pltpu.VMEM((LG, D), jnp.float32),
            pltpu.VMEM((LG, D), jnp.float32),
            pltpu.VMEM((LG, D), jnp.float32),
            pltpu.SemaphoreType.DMA,
            pltpu.SemaphoreType.DMA,
            pltpu.SemaphoreType.DMA,
            pltpu.SemaphoreType.DMA,
            pltpu.SemaphoreType.DMA,
            pltpu.SemaphoreType.DMA,
        ],
    )


def _gather_body(ga, gb, ii, jj, out, idxa, idxb, ra0, ra1, rb0, rb1,
                 sa0, sa1, sb0, sb1, sw0, sw1):
    cid = lax.axis_index("c")
    sid = lax.axis_index("s")
    wid = sid * 2 + cid
    ebase = wid * (NCW * LG)
    nch = jnp.minimum(NCW, NCG - NCW * wid)

    @pl.when(wid < 31)
    def _():
        pltpu.sync_copy(ii.at[pl.ds(ebase, NCW * LG)], idxa)
        pltpu.sync_copy(jj.at[pl.ds(ebase, NCW * LG)], idxb)

    @pl.when(wid == 31)
    def _():
        tail = (NCG - NCW * 31) * LG
        pltpu.sync_copy(ii.at[pl.ds(ebase, tail)], idxa.at[pl.ds(0, tail)])
        pltpu.sync_copy(jj.at[pl.ds(ebase, tail)], idxb.at[pl.ds(0, tail)])

    ra = (ra0, ra1)
    rb = (rb0, rb1)
    sa = (sa0, sa1)
    sb = (sb0, sb1)
    sw = (sw0, sw1)

    def issue(k, par):
        sl = pl.ds(k * LG, LG)
        pltpu.async_copy(ga.at[idxa.at[sl]], ra[par], sa[par])
        pltpu.async_copy(gb.at[idxb.at[sl]], rb[par], sb[par])

    def wait_gathers(k, par):
        sl = pl.ds(k * LG, LG)
        pltpu.make_async_copy(ga.at[idxa.at[sl]], ra[par], sa[par]).wait()
        pltpu.make_async_copy(gb.at[idxb.at[sl]], rb[par], sb[par]).wait()

    def add(par):
        def row(r, c2):
            for c in range(D // 16):
                sl = pl.ds(c * 16, 16)
                plsc.addupdate(ra[par].at[r, sl], rb[par][r, sl])
            return c2

        lax.fori_loop(0, LG, row, 0)

    def issue_wb(k, par):
        pltpu.async_copy(ra[par], out.at[pl.ds(ebase + k * LG, LG)], sw[par])

    def wait_wb(k, par):
        pltpu.make_async_copy(
            ra[par], out.at[pl.ds(ebase + k * LG, LG)], sw[par]).wait()

    issue(0, 0)

    def pair(p, carry):
        k0 = 2 * p
        @pl.when(p > 0)
        def _():
            wait_wb(k0 - 1, 1)

        issue(k0 + 1, 1)
        wait_gathers(k0, 0)
        add(0)
        issue_wb(k0, 0)
        @pl.when(p < nch // 2 - 1)
        def _():
            wait_wb(k0, 0)
            issue(k0 + 2, 0)

        wait_gathers(k0 + 1, 1)
        add(1)
        issue_wb(k0 + 1, 1)
        return carry

    lax.fori_loop(0, nch // 2, pair, 0)
    wait_wb(nch - 2, 0)
    wait_wb(nch - 1, 1)


def _edge_mlp_body(s_ref, ea_ref, w1c_ref, b1_ref, w2_ref, b2_ref, out_ref):
    z = s_ref[:] + jnp.dot(ea_ref[:], w1c_ref[:],
                           preferred_element_type=jnp.float32) + b1_ref[:]
    z = _swish(z)
    m = jnp.dot(z, w2_ref[:], preferred_element_type=jnp.float32) + b2_ref[:]
    out_ref[:] = _swish(m)


def _edge_mlp(s, ea, w1c, b1, w2, b2):
    grid = E // EB
    return pl.pallas_call(
        _edge_mlp_body,
        grid=(grid,),
        in_specs=[
            pl.BlockSpec((EB, D), lambda i: (i, 0)),
            pl.BlockSpec((EB, D), lambda i: (i, 0)),
            pl.BlockSpec((D, D), lambda i: (0, 0)),
            pl.BlockSpec((1, D), lambda i: (0, 0)),
            pl.BlockSpec((D, D), lambda i: (0, 0)),
            pl.BlockSpec((1, D), lambda i: (0, 0)),
        ],
        out_specs=pl.BlockSpec((EB, D), lambda i: (i, 0)),
        out_shape=jax.ShapeDtypeStruct((E, D), jnp.float32),
    )(s, ea, w1c, b1, w2, b2)


NPAD = 10240
_RPT = NPAD // 16


NCD = E // L
NCT = 80


@functools.cache
def _scatter_kernel():
    return pl.kernel(
        _scatter_body,
        out_type=jax.ShapeDtypeStruct((NPAD, D), jnp.float32),
        mesh=_sc_mesh(),
        scratch_types=[
            pltpu.VMEM((L,), jnp.int32),
            pltpu.VMEM((L,), jnp.int32),
            pltpu.VMEM((L, HD), jnp.float32),
            pltpu.VMEM((L, HD), jnp.float32),
            pltpu.VMEM_SHARED((NPAD, HD), jnp.float32),
            pltpu.SemaphoreType.DMA,
            pltpu.SemaphoreType.DMA,
            pltpu.SemaphoreType.DMA,
            pltpu.SemaphoreType.DMA,
        ],
    )


def _scatter_body(mij, ii, out, idx0, idx1, buf0, buf1, acc,
                  sl0, sl1, si0, si1):
    cid = lax.axis_index("c")
    sid = lax.axis_index("s")
    f0 = cid * HD
    base = sid * NCT
    nch = jnp.minimum(NCT, NCD - base)

    zv = jnp.zeros((16,), jnp.float32)

    def zrow(r, c2):
        for c in range(HD // 16):
            buf0[r, pl.ds(c * 16, 16)] = zv
        return c2

    lax.fori_loop(0, L, zrow, 0)
    r0 = sid * _RPT
    for k in range(5):
        pltpu.sync_copy(buf0, acc.at[pl.ds(r0 + k * L, L)])
    plsc.subcore_barrier()

    bufs = (buf0, buf1)
    idxs = (idx0, idx1)
    sls = (sl0, sl1)
    sis = (si0, si1)

    def issue(k, par):
        e0 = (base + k) * L
        pltpu.async_copy(mij.at[pl.ds(e0, L), pl.ds(f0, HD)],
                         bufs[par], sls[par])
        pltpu.async_copy(ii.at[pl.ds(e0, L)], idxs[par], sis[par])

    def wait_load(k, par):
        e0 = (base + k) * L
        pltpu.make_async_copy(mij.at[pl.ds(e0, L), pl.ds(f0, HD)],
                              bufs[par], sls[par]).wait()
        pltpu.make_async_copy(ii.at[pl.ds(e0, L)],
                              idxs[par], sis[par]).wait()

    def scatter_add(par):
        pltpu.sync_copy(bufs[par], acc.at[idxs[par]], add=True)

    issue(0, 0)

    def pair(p, carry):
        k0 = 2 * p
        issue(k0 + 1, 1)
        wait_load(k0, 0)
        scatter_add(0)
        @pl.when(p < nch // 2 - 1)
        def _():
            issue(k0 + 2, 0)

        wait_load(k0 + 1, 1)
        scatter_add(1)
        return carry

    lax.fori_loop(0, nch // 2, pair, 0)
    plsc.subcore_barrier()

    for k in range(5):
        b = bufs[k % 2]
        pltpu.sync_copy(acc.at[pl.ds(r0 + k * L, L)], b)
        pltpu.sync_copy(b, out.at[pl.ds(r0 + k * L, L), pl.ds(f0, HD)])


def _node_mlp_body(h_ref, agg_ref, w1a_ref, w1b_ref, b1_ref, w2_ref, b2_ref,
                   out_ref):
    hb = h_ref[:]
    t = (jnp.dot(hb, w1a_ref[:], preferred_element_type=jnp.float32)
         + jnp.dot(agg_ref[:], w1b_ref[:], preferred_element_type=jnp.float32)
         + b1_ref[:])
    t = _swish(t)
    out_ref[:] = hb + jnp.dot(t, w2_ref[:],
                              preferred_element_type=jnp.float32) + b2_ref[:]


def _node_mlp(h, agg, w1a, w1b, b1, w2, b2):
    grid = N // NB
    return pl.pallas_call(
        _node_mlp_body,
        grid=(grid,),
        in_specs=[
            pl.BlockSpec((NB, D), lambda i: (i, 0)),
            pl.BlockSpec((NB, D), lambda i: (i, 0)),
            pl.BlockSpec((D, D), lambda i: (0, 0)),
            pl.BlockSpec((D, D), lambda i: (0, 0)),
            pl.BlockSpec((1, D), lambda i: (0, 0)),
            pl.BlockSpec((D, D), lambda i: (0, 0)),
            pl.BlockSpec((1, D), lambda i: (0, 0)),
        ],
        out_specs=pl.BlockSpec((NB, D), lambda i: (i, 0)),
        out_shape=jax.ShapeDtypeStruct((N, D), jnp.float32),
    )(h, agg, w1a, w1b, b1, w2, b2)


def kernel(h, edge_index, edge_attr, e_w1, e_b1, e_w2, e_b2,
           n_w1, n_b1, n_w2, n_b2):
    ii = edge_index[0]
    jj = edge_index[1]
    w1a = e_w1[:D]
    w1b = e_w1[D:2 * D]
    w1c = e_w1[2 * D:]

    ga, gb = _node_pre(h, w1a, w1b)
    s = _gather_kernel()(ga, gb, ii, jj)
    mij = _edge_mlp(s, edge_attr, w1c, e_b1.reshape(1, D), e_w2,
                    e_b2.reshape(1, D))
    agg = _scatter_kernel()(mij, ii)
    h_out = _node_mlp(h, agg, n_w1[:D], n_w1[D:], n_b1.reshape(1, D),
                      n_w2, n_b2.reshape(1, D))
    return (h_out, mij)


# speedup vs baseline: 1.1713x; 1.0257x over previous
import functools

import jax
import jax.numpy as jnp
from jax import lax
from jax.experimental import pallas as pl
from jax.experimental.pallas import tpu as pltpu
from jax.experimental.pallas import tpu_sc as plsc

N = 10000
E = 160000
D = 256
HD = D // 2
L = 128
NCH = E // L
NB = 1000
EB = 4000

@functools.cache
def _sc_mesh():
    return plsc.VectorSubcoreMesh(core_axis_name="c", subcore_axis_name="s")


def _swish(x):
    return x * jax.nn.sigmoid(x)


def _node_pre_body(h_ref, wa_ref, wb_ref, ga_ref, gb_ref):
    x = h_ref[:]
    ga_ref[:] = jnp.dot(x, wa_ref[:], preferred_element_type=jnp.float32)
    gb_ref[:] = jnp.dot(x, wb_ref[:], preferred_element_type=jnp.float32)


def _node_pre(h, wa, wb):
    grid = N // NB
    return pl.pallas_call(
        _node_pre_body,
        grid=(grid,),
        in_specs=[
            pl.BlockSpec((NB, D), lambda i: (i, 0)),
            pl.BlockSpec((D, D), lambda i: (0, 0)),
            pl.BlockSpec((D, D), lambda i: (0, 0)),
        ],
        out_specs=[
            pl.BlockSpec((NB, D), lambda i: (i, 0)),
            pl.BlockSpec((NB, D), lambda i: (i, 0)),
        ],
        out_shape=[
            jax.ShapeDtypeStruct((N, D), jnp.float32),
            jax.ShapeDtypeStruct((N, D), jnp.float32),
        ],
    )(h, wa, wb)


LG = 80
NCG = E // LG
NCW = 64


@functools.cache
def _gather_kernel():
    return pl.kernel(
        _gather_body,
        out_type=jax.ShapeDtypeStruct((E, D), jnp.float32),
        mesh=_sc_mesh(),
        scratch_types=[
            pltpu.VMEM((NCW * LG,), jnp.int32),
            pltpu.VMEM((NCW * LG,), jnp.int32),
            pltpu.VMEM((LG, D), jnp.float32),
            pltpu.VMEM((LG, D), jnp.float32),
            pltpu.VMEM((LG, D), jnp.float32),
            pltpu.VMEM((LG, D), jnp.float32),
            pltpu.SemaphoreType.DMA,
            pltpu.SemaphoreType.DMA,
            pltpu.SemaphoreType.DMA,
            pltpu.SemaphoreType.DMA,
            pltpu.SemaphoreType.DMA,
            pltpu.SemaphoreType.DMA,
        ],
    )


def _gather_body(ga, gb, ii, jj, out, idxa, idxb, ra0, ra1, rb0, rb1,
                 sa0, sa1, sb0, sb1, sw0, sw1):
    cid = lax.axis_index("c")
    sid = lax.axis_index("s")
    wid = sid * 2 + cid
    ebase = wid * (NCW * LG)
    nch = jnp.minimum(NCW, NCG - NCW * wid)

    @pl.when(wid < 31)
    def _():
        pltpu.sync_copy(ii.at[pl.ds(ebase, NCW * LG)], idxa)
        pltpu.sync_copy(jj.at[pl.ds(ebase, NCW * LG)], idxb)

    @pl.when(wid == 31)
    def _():
        tail = (NCG - NCW * 31) * LG
        pltpu.sync_copy(ii.at[pl.ds(ebase, tail)], idxa.at[pl.ds(0, tail)])
        pltpu.sync_copy(jj.at[pl.ds(ebase, tail)], idxb.at[pl.ds(0, tail)])

    ra = (ra0, ra1)
    rb = (rb0, rb1)
    sa = (sa0, sa1)
    sb = (sb0, sb1)
    sw = (sw0, sw1)

    def issue(k, par):
        sl = pl.ds(k * LG, LG)
        pltpu.async_copy(ga.at[idxa.at[sl]], ra[par], sa[par])
        pltpu.async_copy(gb.at[idxb.at[sl]], rb[par], sb[par])

    def wait_gathers(k, par):
        sl = pl.ds(k * LG, LG)
        pltpu.make_async_copy(ga.at[idxa.at[sl]], ra[par], sa[par]).wait()
        pltpu.make_async_copy(gb.at[idxb.at[sl]], rb[par], sb[par]).wait()

    def add(par):
        def row(r, c2):
            for c in range(D // 16):
                sl = pl.ds(c * 16, 16)
                plsc.addupdate(ra[par].at[r, sl], rb[par][r, sl])
            return c2

        lax.fori_loop(0, LG, row, 0)

    def issue_wb(k, par):
        pltpu.async_copy(ra[par], out.at[pl.ds(ebase + k * LG, LG)], sw[par])

    def wait_wb(k, par):
        pltpu.make_async_copy(
            ra[par], out.at[pl.ds(ebase + k * LG, LG)], sw[par]).wait()

    issue(0, 0)

    def pair(p, carry):
        k0 = 2 * p
        @pl.when(p > 0)
        def _():
            wait_wb(k0 - 1, 1)

        issue(k0 + 1, 1)
        wait_gathers(k0, 0)
        add(0)
        issue_wb(k0, 0)
        @pl.when(p < nch // 2 - 1)
        def _():
            wait_wb(k0, 0)
            issue(k0 + 2, 0)

        wait_gathers(k0 + 1, 1)
        add(1)
        issue_wb(k0 + 1, 1)
        return carry

    lax.fori_loop(0, nch // 2, pair, 0)
    wait_wb(nch - 2, 0)
    wait_wb(nch - 1, 1)


def _edge_mlp_body(s_ref, ea_ref, w1c_ref, b1_ref, w2_ref, b2_ref, out_ref):
    z = s_ref[:] + jnp.dot(ea_ref[:], w1c_ref[:],
                           preferred_element_type=jnp.float32) + b1_ref[:]
    z = _swish(z)
    m = jnp.dot(z, w2_ref[:], preferred_element_type=jnp.float32) + b2_ref[:]
    out_ref[:] = _swish(m)


def _edge_mlp(s, ea, w1c, b1, w2, b2):
    grid = E // EB
    return pl.pallas_call(
        _edge_mlp_body,
        grid=(grid,),
        in_specs=[
            pl.BlockSpec((EB, D), lambda i: (i, 0)),
            pl.BlockSpec((EB, D), lambda i: (i, 0)),
            pl.BlockSpec((D, D), lambda i: (0, 0)),
            pl.BlockSpec((1, D), lambda i: (0, 0)),
            pl.BlockSpec((D, D), lambda i: (0, 0)),
            pl.BlockSpec((1, D), lambda i: (0, 0)),
        ],
        out_specs=pl.BlockSpec((EB, D), lambda i: (i, 0)),
        out_shape=jax.ShapeDtypeStruct((E, D), jnp.float32),
    )(s, ea, w1c, b1, w2, b2)


NPAD = 10240
_RPT = NPAD // 16


NCD = E // L
NCT = 80


@functools.cache
def _scatter_kernel():
    return pl.kernel(
        _scatter_body,
        out_type=jax.ShapeDtypeStruct((NPAD, D), jnp.float32),
        mesh=_sc_mesh(),
        scratch_types=[
            pltpu.VMEM((L,), jnp.int32),
            pltpu.VMEM((L,), jnp.int32),
            pltpu.VMEM((L, HD), jnp.float32),
            pltpu.VMEM((L, HD), jnp.float32),
            pltpu.VMEM_SHARED((NPAD, HD), jnp.float32),
            pltpu.SemaphoreType.DMA,
            pltpu.SemaphoreType.DMA,
            pltpu.SemaphoreType.DMA,
            pltpu.SemaphoreType.DMA,
        ],
    )


def _scatter_body(mij, ii, out, idx0, idx1, buf0, buf1, acc,
                  sl0, sl1, si0, si1):
    cid = lax.axis_index("c")
    sid = lax.axis_index("s")
    f0 = cid * HD
    base = sid * NCT
    nch = jnp.minimum(NCT, NCD - base)

    zv = jnp.zeros((16,), jnp.float32)

    def zrow(r, c2):
        for c in range(HD // 16):
            buf0[r, pl.ds(c * 16, 16)] = zv
        return c2

    lax.fori_loop(0, L, zrow, 0)
    r0 = sid * _RPT
    for k in range(5):
        pltpu.sync_copy(buf0, acc.at[pl.ds(r0 + k * L, L)])
    plsc.subcore_barrier()

    bufs = (buf0, buf1)
    idxs = (idx0, idx1)
    sls = (sl0, sl1)
    sis = (si0, si1)

    def issue(k, par):
        e0 = (base + k) * L
        pltpu.async_copy(mij.at[pl.ds(e0, L), pl.ds(f0, HD)],
                         bufs[par], sls[par])
        pltpu.async_copy(ii.at[pl.ds(e0, L)], idxs[par], sis[par])

    def wait_load(k, par):
        e0 = (base + k) * L
        pltpu.make_async_copy(mij.at[pl.ds(e0, L), pl.ds(f0, HD)],
                              bufs[par], sls[par]).wait()
        pltpu.make_async_copy(ii.at[pl.ds(e0, L)],
                              idxs[par], sis[par]).wait()

    def scatter_add(par):
        pltpu.sync_copy(bufs[par], acc.at[idxs[par]], add=True)

    issue(0, 0)

    def pair(p, carry):
        k0 = 2 * p
        issue(k0 + 1, 1)
        wait_load(k0, 0)
        scatter_add(0)
        @pl.when(p < nch // 2 - 1)
        def _():
            issue(k0 + 2, 0)

        wait_load(k0 + 1, 1)
        scatter_add(1)
        return carry

    lax.fori_loop(0, nch // 2, pair, 0)
    plsc.subcore_barrier()

    for k in range(5):
        b = bufs[k % 2]
        pltpu.sync_copy(acc.at[pl.ds(r0 + k * L, L)], b)
        pltpu.sync_copy(b, out.at[pl.ds(r0 + k * L, L), pl.ds(f0, HD)])


def _node_mlp_body(h_ref, agg_ref, w1a_ref, w1b_ref, b1_ref, w2_ref, b2_ref,
                   out_ref):
    hb = h_ref[:]
    t = (jnp.dot(hb, w1a_ref[:], preferred_element_type=jnp.float32)
         + jnp.dot(agg_ref[:], w1b_ref[:], preferred_element_type=jnp.float32)
         + b1_ref[:])
    t = _swish(t)
    out_ref[:] = hb + jnp.dot(t, w2_ref[:],
                              preferred_element_type=jnp.float32) + b2_ref[:]


def _node_mlp(h, agg, w1a, w1b, b1, w2, b2):
    grid = N // NB
    return pl.pallas_call(
        _node_mlp_body,
        grid=(grid,),
        in_specs=[
            pl.BlockSpec((NB, D), lambda i: (i, 0)),
            pl.BlockSpec((NB, D), lambda i: (i, 0)),
            pl.BlockSpec((D, D), lambda i: (0, 0)),
            pl.BlockSpec((D, D), lambda i: (0, 0)),
            pl.BlockSpec((1, D), lambda i: (0, 0)),
            pl.BlockSpec((D, D), lambda i: (0, 0)),
            pl.BlockSpec((1, D), lambda i: (0, 0)),
        ],
        out_specs=pl.BlockSpec((NB, D), lambda i: (i, 0)),
        out_shape=jax.ShapeDtypeStruct((N, D), jnp.float32),
    )(h, agg, w1a, w1b, b1, w2, b2)


def kernel(h, edge_index, edge_attr, e_w1, e_b1, e_w2, e_b2,
           n_w1, n_b1, n_w2, n_b2):
    ii = edge_index[0]
    jj = edge_index[1]
    w1a = e_w1[:D]
    w1b = e_w1[D:2 * D]
    w1c = e_w1[2 * D:]

    ga, gb = _node_pre(h, w1a, w1b)
    s = _gather_kernel()(ga, gb, ii, jj)
    mij = _edge_mlp(s, edge_attr, w1c, e_b1.reshape(1, D), e_w2,
                    e_b2.reshape(1, D))
    agg = _scatter_kernel()(mij, ii)
    h_out = _node_mlp(h, agg, n_w1[:D], n_w1[D:], n_b1.reshape(1, D),
                      n_w2, n_b2.reshape(1, D))
    return (h_out, mij)


# speedup vs baseline: 1.1853x; 1.0120x over previous
import functools

import jax
import jax.numpy as jnp
from jax import lax
from jax.experimental import pallas as pl
from jax.experimental.pallas import tpu as pltpu
from jax.experimental.pallas import tpu_sc as plsc

N = 10000
E = 160000
D = 256
HD = D // 2
L = 128
NCH = E // L
NB = 1000
EB = 8000

@functools.cache
def _sc_mesh():
    return plsc.VectorSubcoreMesh(core_axis_name="c", subcore_axis_name="s")


def _swish(x):
    return x * jax.nn.sigmoid(x)


def _node_pre_body(h_ref, wa_ref, wb_ref, ga_ref, gb_ref):
    x = h_ref[:]
    ga_ref[:] = jnp.dot(x, wa_ref[:], preferred_element_type=jnp.float32)
    gb_ref[:] = jnp.dot(x, wb_ref[:], preferred_element_type=jnp.float32)


def _node_pre(h, wa, wb):
    grid = N // NB
    return pl.pallas_call(
        _node_pre_body,
        grid=(grid,),
        in_specs=[
            pl.BlockSpec((NB, D), lambda i: (i, 0)),
            pl.BlockSpec((D, D), lambda i: (0, 0)),
            pl.BlockSpec((D, D), lambda i: (0, 0)),
        ],
        out_specs=[
            pl.BlockSpec((NB, D), lambda i: (i, 0)),
            pl.BlockSpec((NB, D), lambda i: (i, 0)),
        ],
        out_shape=[
            jax.ShapeDtypeStruct((N, D), jnp.float32),
            jax.ShapeDtypeStruct((N, D), jnp.float32),
        ],
    )(h, wa, wb)


LG = 80
NCG = E // LG
NCW = 64


@functools.cache
def _gather_kernel():
    return pl.kernel(
        _gather_body,
        out_type=jax.ShapeDtypeStruct((E, D), jnp.float32),
        mesh=_sc_mesh(),
        scratch_types=[
            pltpu.VMEM((NCW * LG,), jnp.int32),
            pltpu.VMEM((NCW * LG,), jnp.int32),
            pltpu.VMEM((LG, D), jnp.float32),
            pltpu.VMEM((LG, D), jnp.float32),
            pltpu.VMEM((LG, D), jnp.float32),
            pltpu.VMEM((LG, D), jnp.float32),
            pltpu.SemaphoreType.DMA,
            pltpu.SemaphoreType.DMA,
            pltpu.SemaphoreType.DMA,
            pltpu.SemaphoreType.DMA,
            pltpu.SemaphoreType.DMA,
            pltpu.SemaphoreType.DMA,
        ],
    )


def _gather_body(ga, gb, ii, jj, out, idxa, idxb, ra0, ra1, rb0, rb1,
                 sa0, sa1, sb0, sb1, sw0, sw1):
    cid = lax.axis_index("c")
    sid = lax.axis_index("s")
    wid = sid * 2 + cid
    ebase = wid * (NCW * LG)
    nch = jnp.minimum(NCW, NCG - NCW * wid)

    @pl.when(wid < 31)
    def _():
        pltpu.sync_copy(ii.at[pl.ds(ebase, NCW * LG)], idxa)
        pltpu.sync_copy(jj.at[pl.ds(ebase, NCW * LG)], idxb)

    @pl.when(wid == 31)
    def _():
        tail = (NCG - NCW * 31) * LG
        pltpu.sync_copy(ii.at[pl.ds(ebase, tail)], idxa.at[pl.ds(0, tail)])
        pltpu.sync_copy(jj.at[pl.ds(ebase, tail)], idxb.at[pl.ds(0, tail)])

    ra = (ra0, ra1)
    rb = (rb0, rb1)
    sa = (sa0, sa1)
    sb = (sb0, sb1)
    sw = (sw0, sw1)

    def issue(k, par):
        sl = pl.ds(k * LG, LG)
        pltpu.async_copy(ga.at[idxa.at[sl]], ra[par], sa[par])
        pltpu.async_copy(gb.at[idxb.at[sl]], rb[par], sb[par])

    def wait_gathers(k, par):
        sl = pl.ds(k * LG, LG)
        pltpu.make_async_copy(ga.at[idxa.at[sl]], ra[par], sa[par]).wait()
        pltpu.make_async_copy(gb.at[idxb.at[sl]], rb[par], sb[par]).wait()

    def add(par):
        def row(r, c2):
            for c in range(D // 16):
                sl = pl.ds(c * 16, 16)
                plsc.addupdate(ra[par].at[r, sl], rb[par][r, sl])
            return c2

        lax.fori_loop(0, LG, row, 0)

    def issue_wb(k, par):
        pltpu.async_copy(ra[par], out.at[pl.ds(ebase + k * LG, LG)], sw[par])

    def wait_wb(k, par):
        pltpu.make_async_copy(
            ra[par], out.at[pl.ds(ebase + k * LG, LG)], sw[par]).wait()

    issue(0, 0)

    def pair(p, carry):
        k0 = 2 * p
        @pl.when(p > 0)
        def _():
            wait_wb(k0 - 1, 1)

        issue(k0 + 1, 1)
        wait_gathers(k0, 0)
        add(0)
        issue_wb(k0, 0)
        @pl.when(p < nch // 2 - 1)
        def _():
            wait_wb(k0, 0)
            issue(k0 + 2, 0)

        wait_gathers(k0 + 1, 1)
        add(1)
        issue_wb(k0 + 1, 1)
        return carry

    lax.fori_loop(0, nch // 2, pair, 0)
    wait_wb(nch - 2, 0)
    wait_wb(nch - 1, 1)


def _edge_mlp_body(s_ref, ea_ref, w1c_ref, b1_ref, w2_ref, b2_ref, out_ref):
    z = s_ref[:] + jnp.dot(ea_ref[:], w1c_ref[:],
                           preferred_element_type=jnp.float32) + b1_ref[:]
    z = _swish(z)
    m = jnp.dot(z, w2_ref[:], preferred_element_type=jnp.float32) + b2_ref[:]
    out_ref[:] = _swish(m)


def _edge_mlp(s, ea, w1c, b1, w2, b2):
    grid = E // EB
    return pl.pallas_call(
        _edge_mlp_body,
        grid=(grid,),
        in_specs=[
            pl.BlockSpec((EB, D), lambda i: (i, 0)),
            pl.BlockSpec((EB, D), lambda i: (i, 0)),
            pl.BlockSpec((D, D), lambda i: (0, 0)),
            pl.BlockSpec((1, D), lambda i: (0, 0)),
            pl.BlockSpec((D, D), lambda i: (0, 0)),
            pl.BlockSpec((1, D), lambda i: (0, 0)),
        ],
        out_specs=pl.BlockSpec((EB, D), lambda i: (i, 0)),
        out_shape=jax.ShapeDtypeStruct((E, D), jnp.float32),
    )(s, ea, w1c, b1, w2, b2)


NPAD = 10240
_RPT = NPAD // 16


NCD = E // L
NCT = 80


@functools.cache
def _scatter_kernel():
    return pl.kernel(
        _scatter_body,
        out_type=jax.ShapeDtypeStruct((NPAD, D), jnp.float32),
        mesh=_sc_mesh(),
        scratch_types=[
            pltpu.VMEM((L,), jnp.int32),
            pltpu.VMEM((L,), jnp.int32),
            pltpu.VMEM((L, HD), jnp.float32),
            pltpu.VMEM((L, HD), jnp.float32),
            pltpu.VMEM_SHARED((NPAD, HD), jnp.float32),
            pltpu.SemaphoreType.DMA,
            pltpu.SemaphoreType.DMA,
            pltpu.SemaphoreType.DMA,
            pltpu.SemaphoreType.DMA,
        ],
    )


def _scatter_body(mij, ii, out, idx0, idx1, buf0, buf1, acc,
                  sl0, sl1, si0, si1):
    cid = lax.axis_index("c")
    sid = lax.axis_index("s")
    f0 = cid * HD
    base = sid * NCT
    nch = jnp.minimum(NCT, NCD - base)

    zv = jnp.zeros((16,), jnp.float32)

    def zrow(r, c2):
        for c in range(HD // 16):
            buf0[r, pl.ds(c * 16, 16)] = zv
        return c2

    lax.fori_loop(0, L, zrow, 0)
    r0 = sid * _RPT
    for k in range(5):
        pltpu.sync_copy(buf0, acc.at[pl.ds(r0 + k * L, L)])
    plsc.subcore_barrier()

    bufs = (buf0, buf1)
    idxs = (idx0, idx1)
    sls = (sl0, sl1)
    sis = (si0, si1)

    def issue(k, par):
        e0 = (base + k) * L
        pltpu.async_copy(mij.at[pl.ds(e0, L), pl.ds(f0, HD)],
                         bufs[par], sls[par])
        pltpu.async_copy(ii.at[pl.ds(e0, L)], idxs[par], sis[par])

    def wait_load(k, par):
        e0 = (base + k) * L
        pltpu.make_async_copy(mij.at[pl.ds(e0, L), pl.ds(f0, HD)],
                              bufs[par], sls[par]).wait()
        pltpu.make_async_copy(ii.at[pl.ds(e0, L)],
                              idxs[par], sis[par]).wait()

    def scatter_add(par):
        pltpu.sync_copy(bufs[par], acc.at[idxs[par]], add=True)

    issue(0, 0)

    def pair(p, carry):
        k0 = 2 * p
        issue(k0 + 1, 1)
        wait_load(k0, 0)
        scatter_add(0)
        @pl.when(p < nch // 2 - 1)
        def _():
            issue(k0 + 2, 0)

        wait_load(k0 + 1, 1)
        scatter_add(1)
        return carry

    lax.fori_loop(0, nch // 2, pair, 0)
    plsc.subcore_barrier()

    for k in range(5):
        b = bufs[k % 2]
        pltpu.sync_copy(acc.at[pl.ds(r0 + k * L, L)], b)
        pltpu.sync_copy(b, out.at[pl.ds(r0 + k * L, L), pl.ds(f0, HD)])


def _node_mlp_body(h_ref, agg_ref, w1a_ref, w1b_ref, b1_ref, w2_ref, b2_ref,
                   out_ref):
    hb = h_ref[:]
    t = (jnp.dot(hb, w1a_ref[:], preferred_element_type=jnp.float32)
         + jnp.dot(agg_ref[:], w1b_ref[:], preferred_element_type=jnp.float32)
         + b1_ref[:])
    t = _swish(t)
    out_ref[:] = hb + jnp.dot(t, w2_ref[:],
                              preferred_element_type=jnp.float32) + b2_ref[:]


def _node_mlp(h, agg, w1a, w1b, b1, w2, b2):
    grid = N // NB
    return pl.pallas_call(
        _node_mlp_body,
        grid=(grid,),
        in_specs=[
            pl.BlockSpec((NB, D), lambda i: (i, 0)),
            pl.BlockSpec((NB, D), lambda i: (i, 0)),
            pl.BlockSpec((D, D), lambda i: (0, 0)),
            pl.BlockSpec((D, D), lambda i: (0, 0)),
            pl.BlockSpec((1, D), lambda i: (0, 0)),
            pl.BlockSpec((D, D), lambda i: (0, 0)),
            pl.BlockSpec((1, D), lambda i: (0, 0)),
        ],
        out_specs=pl.BlockSpec((NB, D), lambda i: (i, 0)),
        out_shape=jax.ShapeDtypeStruct((N, D), jnp.float32),
    )(h, agg, w1a, w1b, b1, w2, b2)


def kernel(h, edge_index, edge_attr, e_w1, e_b1, e_w2, e_b2,
           n_w1, n_b1, n_w2, n_b2):
    ii = edge_index[0]
    jj = edge_index[1]
    w1a = e_w1[:D]
    w1b = e_w1[D:2 * D]
    w1c = e_w1[2 * D:]

    ga, gb = _node_pre(h, w1a, w1b)
    s = _gather_kernel()(ga, gb, ii, jj)
    mij = _edge_mlp(s, edge_attr, w1c, e_b1.reshape(1, D), e_w2,
                    e_b2.reshape(1, D))
    agg = _scatter_kernel()(mij, ii)
    h_out = _node_mlp(h, agg, n_w1[:D], n_w1[D:], n_b1.reshape(1, D),
                      n_w2, n_b2.reshape(1, D))
    return (h_out, mij)
